# baseline (speedup 1.0000x reference)
import functools

import jax
import jax.numpy as jnp
from jax import lax
from jax.experimental import pallas as pl
from jax.experimental.pallas import tpu as pltpu
from jax.experimental.pallas import tpu_sc as plsc

_N = 10000
_E = 160000
_EMB = 256
_H = 8
_HID = 64
_D = _H * _HID
_NQ = 4
_CW = 128
_BN = 1000
_NT = _N // _BN

_NWORK = 32
_EPAD = 163840
_B = 128
_EPT1 = _EPAD // _NWORK
_NB1 = _EPT1 // _B
_EPT2 = _EPAD // 16
_NB2 = _EPT2 // _B
_RPS = _N // 16

_SC_PARAMS = pltpu.CompilerParams(needs_layout_passes=False,
                                  use_tc_tiling_on_sc=False)


@functools.cache
def _get_mesh():
    return plsc.VectorSubcoreMesh(core_axis_name="c", subcore_axis_name="s",
                                  num_cores=2, num_subcores=16)


def _fold_body(w1, as1, ad1, w2, as2, ad2, ts1, td1, ts2, td2):
    def fold(w, a, o):
        k = w.shape[0]
        f = jnp.sum(w[...].reshape(k, _H, _HID) * a[...][None], axis=-1)
        o[...] = jnp.concatenate([f, jnp.zeros_like(f)], axis=1)
    fold(w1, as1, ts1)
    fold(w1, ad1, td1)
    fold(w2, as2, ts2)
    fold(w2, ad2, td2)


def _fold_weights(W1, a_src1, a_dst1, W2, a_src2, a_dst2):
    return pl.pallas_call(
        _fold_body,
        out_shape=[jax.ShapeDtypeStruct((_EMB, 16), jnp.float32),
                   jax.ShapeDtypeStruct((_EMB, 16), jnp.float32),
                   jax.ShapeDtypeStruct((_D, 16), jnp.float32),
                   jax.ShapeDtypeStruct((_D, 16), jnp.float32)],
    )(W1, a_src1, a_dst1, W2, a_src2, a_dst2)


def _ilv_bf16(x):
    return (x.reshape(_BN, 4, 2, 16).swapaxes(2, 3)
            .reshape(_BN, _CW).astype(jnp.bfloat16))


def _l1_body(z, wq, ws, wd, xp, ts, td):
    zb = z[...]
    xp[...] = jnp.dot(zb, wq[...], preferred_element_type=jnp.float32)
    ts[...] = jnp.dot(zb, ws[...], preferred_element_type=jnp.float32)
    td[...] = jnp.dot(zb, wd[...], preferred_element_type=jnp.float32)


def _layer1_tc(z, W1, Ws1, Wd1):
    return pl.pallas_call(
        _l1_body,
        grid=(_NT, _NQ),
        in_specs=[
            pl.BlockSpec((_BN, _EMB), lambda i, q: (i, 0)),
            pl.BlockSpec((_EMB, _CW), lambda i, q: (0, q)),
            pl.BlockSpec((_EMB, 16), lambda i, q: (0, 0)),
            pl.BlockSpec((_EMB, 16), lambda i, q: (0, 0)),
        ],
        out_specs=[
            pl.BlockSpec((_BN, _CW), lambda i, q: (q * _NT + i, 0)),
            pl.BlockSpec((_BN, 16), lambda i, q: (i, 0)),
            pl.BlockSpec((_BN, 16), lambda i, q: (i, 0)),
        ],
        out_shape=[jax.ShapeDtypeStruct((_NQ * _N, _CW), jnp.float32),
                   jax.ShapeDtypeStruct((_N, 16), jnp.float32),
                   jax.ShapeDtypeStruct((_N, 16), jnp.float32)],
        compiler_params=pltpu.CompilerParams(
            dimension_semantics=("arbitrary", "arbitrary")),
    )(z, W1, Ws1, Wd1)


def _bf_body(xp, xpb):
    xpb[...] = _ilv_bf16(xp[...])


def _to_bf16_ilv(xp4):
    return pl.pallas_call(
        _bf_body,
        grid=(_NQ * _NT,),
        in_specs=[pl.BlockSpec((_BN, _CW), lambda i: (i, 0))],
        out_specs=pl.BlockSpec((_BN, _CW), lambda i: (i, 0)),
        out_shape=jax.ShapeDtypeStruct((_NQ * _N, _CW), jnp.bfloat16),
        compiler_params=pltpu.CompilerParams(
            dimension_semantics=("arbitrary",)),
    )(xp4)


def _finalize_block(ou_refs, xp_refs, dp_ref, ts_ref, td_ref, b_ref):
    dsum = jnp.sum(dp_ref[...], axis=0)
    al = ts_ref[...][:, :_H] + td_ref[...][:, :_H]
    el = jnp.exp(jnp.maximum(al, 0.2 * al))
    dtot = dsum + el + 1e-16
    bb = b_ref[...]
    parts = []
    for q in range(_NQ):
        ou = ou_refs[q][...]
        xp = xp_refs[q][...]
        e2 = jnp.concatenate(
            [jnp.broadcast_to(el[:, 2 * q:2 * q + 1], (_BN, _HID)),
             jnp.broadcast_to(el[:, 2 * q + 1:2 * q + 2], (_BN, _HID))], axis=1)
        d2 = jnp.concatenate(
            [jnp.broadcast_to(dtot[:, 2 * q:2 * q + 1], (_BN, _HID)),
             jnp.broadcast_to(dtot[:, 2 * q + 1:2 * q + 2], (_BN, _HID))], axis=1)
        parts.append(jnp.tanh((ou + e2 * xp) / d2 + bb[:, q * _CW:(q + 1) * _CW]))
    return jnp.concatenate(parts, axis=1)


def _l2_body(ou0, ou1, ou2, ou3, xq0, xq1, xq2, xq3, dp, ts, td, b, wq, ws, wd,
             xp2, ts2, td2):
    x2 = _finalize_block((ou0, ou1, ou2, ou3), (xq0, xq1, xq2, xq3),
                         dp, ts, td, b)
    xp2[...] = jnp.dot(x2, wq[...], preferred_element_type=jnp.float32)
    ts2[...] = jnp.dot(x2, ws[...], preferred_element_type=jnp.float32)
    td2[...] = jnp.dot(x2, wd[...], preferred_element_type=jnp.float32)


def _layer2_tc(ou4, xp4, dparts, Ts1, Td1, b1r, W2, Ws2, Wd2):
    chunk = lambda q: pl.BlockSpec((_BN, _CW), lambda i, co, q=q: (q * _NT + i, 0))
    return pl.pallas_call(
        _l2_body,
        grid=(_NT, _NQ),
        in_specs=[chunk(0), chunk(1), chunk(2), chunk(3),
                  chunk(0), chunk(1), chunk(2), chunk(3),
                  pl.BlockSpec((_NWORK, _BN, _H), lambda i, co: (0, i, 0)),
                  pl.BlockSpec((_BN, 16), lambda i, co: (i, 0)),
                  pl.BlockSpec((_BN, 16), lambda i, co: (i, 0)),
                  pl.BlockSpec((1, _D), lambda i, co: (0, 0)),
                  pl.BlockSpec((_D, _CW), lambda i, co: (0, co)),
                  pl.BlockSpec((_D, 16), lambda i, co: (0, 0)),
                  pl.BlockSpec((_D, 16), lambda i, co: (0, 0))],
        out_specs=[
            pl.BlockSpec((_BN, _CW), lambda i, co: (co * _NT + i, 0)),
            pl.BlockSpec((_BN, 16), lambda i, co: (i, 0)),
            pl.BlockSpec((_BN, 16), lambda i, co: (i, 0)),
        ],
        out_shape=[jax.ShapeDtypeStruct((_NQ * _N, _CW), jnp.float32),
                   jax.ShapeDtypeStruct((_N, 16), jnp.float32),
                   jax.ShapeDtypeStruct((_N, 16), jnp.float32)],
        compiler_params=pltpu.CompilerParams(
            dimension_semantics=("arbitrary", "arbitrary")),
    )(ou4, ou4, ou4, ou4, xp4, xp4, xp4, xp4, dparts, Ts1, Td1, b1r, W2, Ws2, Wd2)


def _l3_body(ou0, ou1, ou2, ou3, xq0, xq1, xq2, xq3, dp, ts, td, b, wl, bl, out):
    x3 = _finalize_block((ou0, ou1, ou2, ou3), (xq0, xq1, xq2, xq3),
                         dp, ts, td, b)
    out[...] = jnp.dot(x3, wl[...], preferred_element_type=jnp.float32) + bl[...]


def _layer3_tc(ou4, xp4, dparts, Ts2, Td2, b2r, Wlin, blinr):
    chunk = lambda q: pl.BlockSpec((_BN, _CW), lambda i, q=q: (q * _NT + i, 0))
    return pl.pallas_call(
        _l3_body,
        grid=(_NT,),
        in_specs=[chunk(0), chunk(1), chunk(2), chunk(3),
                  chunk(0), chunk(1), chunk(2), chunk(3),
                  pl.BlockSpec((_NWORK, _BN, _H), lambda i: (0, i, 0)),
                  pl.BlockSpec((_BN, 16), lambda i: (i, 0)),
                  pl.BlockSpec((_BN, 16), lambda i: (i, 0)),
                  pl.BlockSpec((1, _D), lambda i: (0, 0)),
                  pl.BlockSpec((_D, 1), lambda i: (0, 0)),
                  pl.BlockSpec((1, 1), lambda i: (0, 0))],
        out_specs=pl.BlockSpec((_BN, 1), lambda i: (i, 0)),
        out_shape=jax.ShapeDtypeStruct((_N, 1), jnp.float32),
        compiler_params=pltpu.CompilerParams(
            dimension_semantics=("arbitrary",)),
    )(ou4, ou4, ou4, ou4, xp4, xp4, xp4, xp4, dparts, Ts2, Td2, b2r, Wlin, blinr)


def _sc_pass1_body(srcp, dstp, ts, td, zer, eeT, dparts,
                   src_v, dst_v, g1, g2, eeb, den_v, gsem, ssem):
    w = lax.axis_index("s") * 2 + lax.axis_index("c")
    lane = lax.iota(jnp.int32, 16)
    m8 = lane < 8
    pltpu.sync_copy(zer, den_v)
    pltpu.sync_copy(srcp.at[pl.ds(w * _NB1, _NB1), :], src_v)
    pltpu.sync_copy(dstp.at[pl.ds(w * _NB1, _NB1), :], dst_v)

    def group(g, _):
        def gathers(b, slot):
            j = g * _GRP + b
            return (pltpu.async_copy(ts.at[src_v.at[j]], g1.at[slot], gsem),
                    pltpu.async_copy(td.at[dst_v.at[j]], g2.at[slot], gsem))

        pend_g = gathers(0, 0)
        pend_s = [None, None]
        for b in range(_GRP):
            sl = b % 2
            j = g * _GRP + b
            for d in pend_g:
                d.wait()
            if b + 1 < _GRP:
                pend_g = gathers(b + 1, (b + 1) % 2)
            if pend_s[sl] is not None:
                for d in pend_s[sl]:
                    d.wait()
                pend_s[sl] = None
            base = (w * _NB1 + j) * _B
            j16 = jnp.full((16,), j, jnp.int32)
            s16 = jnp.full((16,), sl, jnp.int32)

            def edge(k, _):
                k16 = jnp.full((16,), k, jnp.int32)
                v = g1[sl, k, :] + g2[sl, k, :]
                ee = jnp.exp(jnp.maximum(v, 0.2 * v))
                ee = jnp.where(base + k < _E, ee, 0.0)
                dk = plsc.load_gather(dst_v, [j16, k16])
                plsc.addupdate_scatter(den_v, [dk, lane], ee, mask=m8)
                plsc.store_scatter(eeb, [s16, lane * _B + k16], ee, mask=m8)
                return 0

            lax.fori_loop(0, _B, edge, 0)
            pend_s[sl] = tuple(
                pltpu.async_copy(eeb.at[sl, pl.ds(h * _B, _B)],
                                 eeT.at[h, w * _NB1 + j], ssem)
                for h in range(_H))
        for p in pend_s:
            if p is not None:
                for d in p:
                    d.wait()
        return 0

    lax.fori_loop(0, _NB1 // _GRP, group, 0)
    pltpu.sync_copy(den_v, dparts.at[w])


def _sc_pass1(srcp, dstp, Ts, Td, zeros_n8):
    return pl.kernel(
        _sc_pass1_body,
        out_type=[jax.ShapeDtypeStruct((_H, _EPAD // _B, _B), jnp.float32),
                  jax.ShapeDtypeStruct((_NWORK, _N, _H), jnp.float32)],
        mesh=_get_mesh(),
        compiler_params=_SC_PARAMS,
        scratch_types=[
            pltpu.VMEM((_NB1, _B), jnp.int32),
            pltpu.VMEM((_NB1, _B), jnp.int32),
            pltpu.VMEM((2, _B, 16), jnp.float32),
            pltpu.VMEM((2, _B, 16), jnp.float32),
            pltpu.VMEM((2, _H * _B), jnp.float32),
            pltpu.VMEM((_N, _H), jnp.float32),
            pltpu.SemaphoreType.DMA,
            pltpu.SemaphoreType.DMA,
        ],
    )(srcp, dstp, Ts, Td, zeros_n8)


_GRP = 8
_GRP2 = 16


def _sc_pass2_body(srcp, dstp, eeT, xpb, zer, ou4,
                   src_v, dst_v, idx_v, eeb, rbf, msg,
                   acc, lsem, gsem, ssem):
    c = lax.axis_index("c")
    s = lax.axis_index("s")
    pltpu.sync_copy(zer, acc.at[pl.ds(s * _RPS, _RPS), :])
    plsc.subcore_barrier()
    for qi in range(2):
        q = 2 * c + qi
        qn = q * _N

        def group(g, _):
            row0 = s * _NB2 + g * _GRP2

            def loads(b):
                r = row0 + b
                return (pltpu.async_copy(srcp.at[r], src_v.at[b % 3], lsem),
                        pltpu.async_copy(dstp.at[r], dst_v.at[b % 3], lsem),
                        pltpu.async_copy(eeT.at[pl.ds(2 * q, 2), r],
                                         eeb.at[b % 3], lsem))

            def build_and_gather(b):
                for l in range(_B // 16):
                    idx_v[b % 2, pl.ds(l * 16, 16)] = (
                        src_v[b % 3, pl.ds(l * 16, 16)] + qn)
                return pltpu.async_copy(xpb.at[idx_v.at[b % 2]],
                                        rbf.at[b % 2], gsem)

            pend_l = loads(0)
            for d in pend_l:
                d.wait()
            pend_g = [build_and_gather(0), None]
            pend_l = loads(1)
            pend_s = [None, None]
            for b in range(_GRP2):
                s2, s3 = b % 2, b % 3
                n2 = (b + 1) % 2
                if b + 1 < _GRP2:
                    for d in pend_l:
                        d.wait()
                    if pend_s[n2] is not None:
                        pend_s[n2].wait()
                        pend_s[n2] = None
                    pend_g[n2] = build_and_gather(b + 1)
                    if b + 2 < _GRP2:
                        pend_l = loads(b + 2)
                pend_g[s2].wait()
                pend_g[s2] = None

                s16 = jnp.full((16,), s3, jnp.int32)
                h0 = jnp.zeros((16,), jnp.int32)
                h1 = jnp.ones((16,), jnp.int32)

                def edge(k, _):
                    k16 = jnp.full((16,), k, jnp.int32)
                    e0 = plsc.load_gather(eeb, [s16, h0, k16])
                    e1 = plsc.load_gather(eeb, [s16, h1, k16])
                    for gblk in range(4):
                        v = rbf[s2, k, pl.ds(gblk * 32, 32)]
                        lo, hi = plsc.unpack(
                            v, format=plsc.PackFormat.INTERLEAVED)
                        e = e0 if gblk < 2 else e1
                        msg[s2, k, pl.ds(gblk * 32, 16)] = lo * e
                        msg[s2, k, pl.ds(gblk * 32 + 16, 16)] = hi * e
                    return 0

                lax.fori_loop(0, _B, edge, 0)
                pend_s[s2] = pltpu.async_copy(msg.at[s2],
                                              acc.at[dst_v.at[s3]], ssem,
                                              add=True)
            for d in pend_s:
                if d is not None:
                    d.wait()
            return 0

        lax.fori_loop(0, _NB2 // _GRP2, group, 0)
        plsc.subcore_barrier()
        pltpu.sync_copy(acc.at[pl.ds(s * _RPS, _RPS), :],
                        ou4.at[pl.ds(qn + s * _RPS, _RPS), :])
        plsc.subcore_barrier()
        if qi == 0:
            pltpu.sync_copy(zer, acc.at[pl.ds(s * _RPS, _RPS), :])
            plsc.subcore_barrier()


def _sc_pass2(srcp, dstp, eeT, xpb, zeros_acc):
    return pl.kernel(
        _sc_pass2_body,
        out_type=jax.ShapeDtypeStruct((_NQ * _N, _CW), jnp.float32),
        mesh=_get_mesh(),
        compiler_params=_SC_PARAMS,
        scratch_types=[
            pltpu.VMEM((3, _B), jnp.int32),
            pltpu.VMEM((3, _B), jnp.int32),
            pltpu.VMEM((2, _B), jnp.int32),
            pltpu.VMEM((3, 2, _B), jnp.float32),
            pltpu.VMEM((2, _B, _CW), jnp.bfloat16),
            pltpu.VMEM((2, _B, _CW), jnp.float32),
            pltpu.VMEM_SHARED((_N, _CW), jnp.float32),
            pltpu.SemaphoreType.DMA,
            pltpu.SemaphoreType.DMA,
            pltpu.SemaphoreType.DMA,
        ],
    )(srcp, dstp, eeT, xpb, zeros_acc)


def kernel(z, edge_index, W1, a_src1, a_dst1, b1, W2, a_src2, a_dst2, b2,
           Wlin, blin):
    ei = edge_index.astype(jnp.int32)
    pad = jnp.zeros((_EPAD - _E,), jnp.int32)
    srcp = jnp.concatenate([ei[0], pad]).reshape(_EPAD // _B, _B)
    dstp = jnp.concatenate([ei[1], pad]).reshape(_EPAD // _B, _B)
    zeros_n8 = jnp.zeros((_N, _H), jnp.float32)
    zeros_acc = jnp.zeros((_RPS, _CW), jnp.float32)
    b1r = b1.reshape(1, _D)
    b2r = b2.reshape(1, _D)
    blinr = blin.reshape(1, 1)

    Ts1w, Td1w, Ts2w, Td2w = _fold_weights(W1, a_src1, a_dst1, W2, a_src2, a_dst2)

    xp4_1, Ts1, Td1 = _layer1_tc(z, W1, Ts1w, Td1w)
    xpb_1 = _to_bf16_ilv(xp4_1)
    eeT1, dparts1 = _sc_pass1(srcp, dstp, Ts1, Td1, zeros_n8)
    ou4_1 = _sc_pass2(srcp, dstp, eeT1, xpb_1, zeros_acc)

    xp4_2, Ts2, Td2 = _layer2_tc(ou4_1, xp4_1, dparts1, Ts1, Td1, b1r,
                                 W2, Ts2w, Td2w)
    xpb_2 = _to_bf16_ilv(xp4_2)
    eeT2, dparts2 = _sc_pass1(srcp, dstp, Ts2, Td2, zeros_n8)
    ou4_2 = _sc_pass2(srcp, dstp, eeT2, xpb_2, zeros_acc)

    return _layer3_tc(ou4_2, xp4_2, dparts2, Ts2, Td2, b2r, Wlin, blinr)


# speedup vs baseline: 1.7220x; 1.7220x over previous
import functools

import jax
import jax.numpy as jnp
from jax import lax
from jax.experimental import pallas as pl
from jax.experimental.pallas import tpu as pltpu
from jax.experimental.pallas import tpu_sc as plsc

_N = 10000
_E = 160000
_EMB = 256
_H = 8
_HID = 64
_D = _H * _HID
_NQ = 4
_CW = 128
_BN = 1000
_NT = _N // _BN

_NWORK = 32
_EPAD = 163840
_B = 128
_EPT1 = _EPAD // _NWORK
_NB1 = _EPT1 // _B
_EPT2 = _EPAD // 16
_NB2 = _EPT2 // _B
_RPS = _N // 16

_SC_PARAMS = pltpu.CompilerParams(needs_layout_passes=False,
                                  use_tc_tiling_on_sc=False)


@functools.cache
def _get_mesh():
    return plsc.VectorSubcoreMesh(core_axis_name="c", subcore_axis_name="s",
                                  num_cores=2, num_subcores=16)


def _fold_body(w1, as1, ad1, w2, as2, ad2, ts1, td1, ts2, td2):
    def fold(w, a, o):
        k = w.shape[0]
        f = jnp.sum(w[...].reshape(k, _H, _HID) * a[...][None], axis=-1)
        o[...] = jnp.concatenate([f, jnp.zeros_like(f)], axis=1)
    fold(w1, as1, ts1)
    fold(w1, ad1, td1)
    fold(w2, as2, ts2)
    fold(w2, ad2, td2)


def _fold_weights(W1, a_src1, a_dst1, W2, a_src2, a_dst2):
    return pl.pallas_call(
        _fold_body,
        out_shape=[jax.ShapeDtypeStruct((_EMB, 16), jnp.float32),
                   jax.ShapeDtypeStruct((_EMB, 16), jnp.float32),
                   jax.ShapeDtypeStruct((_D, 16), jnp.float32),
                   jax.ShapeDtypeStruct((_D, 16), jnp.float32)],
    )(W1, a_src1, a_dst1, W2, a_src2, a_dst2)


def _l1_body(z, wq, ws, wd, xp, ts, td):
    zb = z[...]
    xp[...] = jnp.dot(zb, wq[...], preferred_element_type=jnp.float32)
    ts[...] = jnp.dot(zb, ws[...], preferred_element_type=jnp.float32)
    td[...] = jnp.dot(zb, wd[...], preferred_element_type=jnp.float32)


def _layer1_tc(z, W1, Ws1, Wd1):
    return pl.pallas_call(
        _l1_body,
        grid=(_NT, _NQ),
        in_specs=[
            pl.BlockSpec((_BN, _EMB), lambda i, q: (i, 0)),
            pl.BlockSpec((_EMB, _CW), lambda i, q: (0, q)),
            pl.BlockSpec((_EMB, 16), lambda i, q: (0, 0)),
            pl.BlockSpec((_EMB, 16), lambda i, q: (0, 0)),
        ],
        out_specs=[
            pl.BlockSpec((_BN, _CW), lambda i, q: (q * _NT + i, 0)),
            pl.BlockSpec((_BN, 16), lambda i, q: (i, 0)),
            pl.BlockSpec((_BN, 16), lambda i, q: (i, 0)),
        ],
        out_shape=[jax.ShapeDtypeStruct((_NQ * _N, _CW), jnp.float32),
                   jax.ShapeDtypeStruct((_N, 16), jnp.float32),
                   jax.ShapeDtypeStruct((_N, 16), jnp.float32)],
        compiler_params=pltpu.CompilerParams(
            dimension_semantics=("arbitrary", "arbitrary")),
    )(z, W1, Ws1, Wd1)


def _finalize_block(ou_refs, xp_refs, dp_ref, ts_ref, td_ref, b_ref):
    dsum = jnp.sum(dp_ref[...], axis=0)
    al = ts_ref[...][:, :_H] + td_ref[...][:, :_H]
    el = jnp.exp(jnp.maximum(al, 0.2 * al))
    dtot = dsum + el + 1e-16
    bb = b_ref[...]
    parts = []
    for q in range(_NQ):
        ou = ou_refs[q][...]
        xp = xp_refs[q][...]
        e2 = jnp.concatenate(
            [jnp.broadcast_to(el[:, 2 * q:2 * q + 1], (_BN, _HID)),
             jnp.broadcast_to(el[:, 2 * q + 1:2 * q + 2], (_BN, _HID))], axis=1)
        d2 = jnp.concatenate(
            [jnp.broadcast_to(dtot[:, 2 * q:2 * q + 1], (_BN, _HID)),
             jnp.broadcast_to(dtot[:, 2 * q + 1:2 * q + 2], (_BN, _HID))], axis=1)
        parts.append(jnp.tanh((ou + e2 * xp) / d2 + bb[:, q * _CW:(q + 1) * _CW]))
    return jnp.concatenate(parts, axis=1)


def _l2_body(ou0, ou1, ou2, ou3, xq0, xq1, xq2, xq3, dp, ts, td, b, wq, ws, wd,
             xp2, ts2, td2):
    x2 = _finalize_block((ou0, ou1, ou2, ou3), (xq0, xq1, xq2, xq3),
                         dp, ts, td, b)
    xp2[...] = jnp.dot(x2, wq[...], preferred_element_type=jnp.float32)
    ts2[...] = jnp.dot(x2, ws[...], preferred_element_type=jnp.float32)
    td2[...] = jnp.dot(x2, wd[...], preferred_element_type=jnp.float32)


def _layer2_tc(ou4, xp4, dparts, Ts1, Td1, b1r, W2, Ws2, Wd2):
    chunk = lambda q: pl.BlockSpec((_BN, _CW), lambda i, co, q=q: (q * _NT + i, 0))
    return pl.pallas_call(
        _l2_body,
        grid=(_NT, _NQ),
        in_specs=[chunk(0), chunk(1), chunk(2), chunk(3),
                  chunk(0), chunk(1), chunk(2), chunk(3),
                  pl.BlockSpec((_NWORK, _BN, _H), lambda i, co: (0, i, 0)),
                  pl.BlockSpec((_BN, 16), lambda i, co: (i, 0)),
                  pl.BlockSpec((_BN, 16), lambda i, co: (i, 0)),
                  pl.BlockSpec((1, _D), lambda i, co: (0, 0)),
                  pl.BlockSpec((_D, _CW), lambda i, co: (0, co)),
                  pl.BlockSpec((_D, 16), lambda i, co: (0, 0)),
                  pl.BlockSpec((_D, 16), lambda i, co: (0, 0))],
        out_specs=[
            pl.BlockSpec((_BN, _CW), lambda i, co: (co * _NT + i, 0)),
            pl.BlockSpec((_BN, 16), lambda i, co: (i, 0)),
            pl.BlockSpec((_BN, 16), lambda i, co: (i, 0)),
        ],
        out_shape=[jax.ShapeDtypeStruct((_NQ * _N, _CW), jnp.float32),
                   jax.ShapeDtypeStruct((_N, 16), jnp.float32),
                   jax.ShapeDtypeStruct((_N, 16), jnp.float32)],
        compiler_params=pltpu.CompilerParams(
            dimension_semantics=("arbitrary", "arbitrary")),
    )(ou4, ou4, ou4, ou4, xp4, xp4, xp4, xp4, dparts, Ts1, Td1, b1r, W2, Ws2, Wd2)


def _l3_body(ou0, ou1, ou2, ou3, xq0, xq1, xq2, xq3, dp, ts, td, b, wl, bl, out):
    x3 = _finalize_block((ou0, ou1, ou2, ou3), (xq0, xq1, xq2, xq3),
                         dp, ts, td, b)
    out[...] = jnp.dot(x3, wl[...], preferred_element_type=jnp.float32) + bl[...]


def _layer3_tc(ou4, xp4, dparts, Ts2, Td2, b2r, Wlin, blinr):
    chunk = lambda q: pl.BlockSpec((_BN, _CW), lambda i, q=q: (q * _NT + i, 0))
    return pl.pallas_call(
        _l3_body,
        grid=(_NT,),
        in_specs=[chunk(0), chunk(1), chunk(2), chunk(3),
                  chunk(0), chunk(1), chunk(2), chunk(3),
                  pl.BlockSpec((_NWORK, _BN, _H), lambda i: (0, i, 0)),
                  pl.BlockSpec((_BN, 16), lambda i: (i, 0)),
                  pl.BlockSpec((_BN, 16), lambda i: (i, 0)),
                  pl.BlockSpec((1, _D), lambda i: (0, 0)),
                  pl.BlockSpec((_D, 1), lambda i: (0, 0)),
                  pl.BlockSpec((1, 1), lambda i: (0, 0))],
        out_specs=pl.BlockSpec((_BN, 1), lambda i: (i, 0)),
        out_shape=jax.ShapeDtypeStruct((_N, 1), jnp.float32),
        compiler_params=pltpu.CompilerParams(
            dimension_semantics=("arbitrary",)),
    )(ou4, ou4, ou4, ou4, xp4, xp4, xp4, xp4, dparts, Ts2, Td2, b2r, Wlin, blinr)


def _sc_pass1_body(srcp, dstp, ts, td, zer, eeT, dparts,
                   src_v, dst_v, g1, g2, eeb, den_v, gsem, ssem):
    w = lax.axis_index("s") * 2 + lax.axis_index("c")
    lane = lax.iota(jnp.int32, 16)
    m8 = lane < 8
    pltpu.sync_copy(zer, den_v)
    pltpu.sync_copy(srcp.at[pl.ds(w * _NB1, _NB1), :], src_v)
    pltpu.sync_copy(dstp.at[pl.ds(w * _NB1, _NB1), :], dst_v)

    def group(g, _):
        def gathers(b, slot):
            j = g * _GRP + b
            return (pltpu.async_copy(ts.at[src_v.at[j]], g1.at[slot], gsem),
                    pltpu.async_copy(td.at[dst_v.at[j]], g2.at[slot], gsem))

        pend_g = gathers(0, 0)
        pend_s = [None, None]
        for b in range(_GRP):
            sl = b % 2
            j = g * _GRP + b
            for d in pend_g:
                d.wait()
            if b + 1 < _GRP:
                pend_g = gathers(b + 1, (b + 1) % 2)
            if pend_s[sl] is not None:
                for d in pend_s[sl]:
                    d.wait()
                pend_s[sl] = None
            base = (w * _NB1 + j) * _B
            j16 = jnp.full((16,), j, jnp.int32)
            s16 = jnp.full((16,), sl, jnp.int32)

            def edge(k, _):
                k16 = jnp.full((16,), k, jnp.int32)
                v = g1[sl, k, :] + g2[sl, k, :]
                ee = jnp.exp(jnp.maximum(v, 0.2 * v))
                ee = jnp.where(base + k < _E, ee, 0.0)
                dk = plsc.load_gather(dst_v, [j16, k16])
                plsc.addupdate_scatter(den_v, [dk, lane], ee, mask=m8)
                plsc.store_scatter(eeb, [s16, lane * _B + k16], ee, mask=m8)
                return 0

            lax.fori_loop(0, _B, edge, 0)
            pend_s[sl] = tuple(
                pltpu.async_copy(eeb.at[sl, pl.ds(h * _B, _B)],
                                 eeT.at[h, w * _NB1 + j], ssem)
                for h in range(_H))
        for p in pend_s:
            if p is not None:
                for d in p:
                    d.wait()
        return 0

    lax.fori_loop(0, _NB1 // _GRP, group, 0)
    pltpu.sync_copy(den_v, dparts.at[w])


def _sc_pass1(srcp, dstp, Ts, Td, zeros_n8):
    return pl.kernel(
        _sc_pass1_body,
        out_type=[jax.ShapeDtypeStruct((_H, _EPAD // _B, _B), jnp.float32),
                  jax.ShapeDtypeStruct((_NWORK, _N, _H), jnp.float32)],
        mesh=_get_mesh(),
        compiler_params=_SC_PARAMS,
        scratch_types=[
            pltpu.VMEM((_NB1, _B), jnp.int32),
            pltpu.VMEM((_NB1, _B), jnp.int32),
            pltpu.VMEM((2, _B, 16), jnp.float32),
            pltpu.VMEM((2, _B, 16), jnp.float32),
            pltpu.VMEM((2, _H * _B), jnp.float32),
            pltpu.VMEM((_N, _H), jnp.float32),
            pltpu.SemaphoreType.DMA,
            pltpu.SemaphoreType.DMA,
        ],
    )(srcp, dstp, Ts, Td, zeros_n8)


_GRP = 8
_GRP2 = 16


def _sc_pass2_body(srcp, dstp, eeT, xp4, zer, ou4,
                   src_v, dst_v, idx_v, eeb, rows,
                   acc, lsem, gsem, ssem):
    c = lax.axis_index("c")
    s = lax.axis_index("s")
    pltpu.sync_copy(zer, acc.at[pl.ds(s * _RPS, _RPS), :])
    plsc.subcore_barrier()
    for qi in range(2):
        q = 2 * c + qi
        qn = q * _N

        def group(g, _):
            row0 = s * _NB2 + g * _GRP2

            def loads(b):
                r = row0 + b
                return (pltpu.async_copy(srcp.at[r], src_v.at[b % 3], lsem),
                        pltpu.async_copy(dstp.at[r], dst_v.at[b % 3], lsem),
                        pltpu.async_copy(eeT.at[pl.ds(2 * q, 2), r],
                                         eeb.at[b % 3], lsem))

            def build_and_gather(b):
                for l in range(_B // 16):
                    idx_v[b % 2, pl.ds(l * 16, 16)] = (
                        src_v[b % 3, pl.ds(l * 16, 16)] + qn)
                return pltpu.async_copy(xp4.at[idx_v.at[b % 2]],
                                        rows.at[b % 2], gsem)

            pend_l = loads(0)
            for d in pend_l:
                d.wait()
            pend_g = [build_and_gather(0), None]
            pend_l = loads(1)
            pend_s = [None, None]
            for b in range(_GRP2):
                s2, s3 = b % 2, b % 3
                n2 = (b + 1) % 2
                if b + 1 < _GRP2:
                    for d in pend_l:
                        d.wait()
                    if pend_s[n2] is not None:
                        pend_s[n2].wait()
                        pend_s[n2] = None
                    pend_g[n2] = build_and_gather(b + 1)
                    if b + 2 < _GRP2:
                        pend_l = loads(b + 2)
                pend_g[s2].wait()
                pend_g[s2] = None

                s16 = jnp.full((16,), s3, jnp.int32)
                h0 = jnp.zeros((16,), jnp.int32)
                h1 = jnp.ones((16,), jnp.int32)

                def edge(i, _):
                    k = 2 * i
                    k16 = jnp.full((16,), k, jnp.int32)
                    e0a = plsc.load_gather(eeb, [s16, h0, k16])
                    e1a = plsc.load_gather(eeb, [s16, h1, k16])
                    e0b = plsc.load_gather(eeb, [s16, h0, k16 + 1])
                    e1b = plsc.load_gather(eeb, [s16, h1, k16 + 1])
                    for l in range(4):
                        rows[s2, k, pl.ds(l * 16, 16)] = (
                            rows[s2, k, pl.ds(l * 16, 16)] * e0a)
                        rows[s2, k + 1, pl.ds(l * 16, 16)] = (
                            rows[s2, k + 1, pl.ds(l * 16, 16)] * e0b)
                    for l in range(4, 8):
                        rows[s2, k, pl.ds(l * 16, 16)] = (
                            rows[s2, k, pl.ds(l * 16, 16)] * e1a)
                        rows[s2, k + 1, pl.ds(l * 16, 16)] = (
                            rows[s2, k + 1, pl.ds(l * 16, 16)] * e1b)
                    return 0

                lax.fori_loop(0, _B // 2, edge, 0)
                pend_s[s2] = pltpu.async_copy(rows.at[s2],
                                              acc.at[dst_v.at[s3]], ssem,
                                              add=True)
            for d in pend_s:
                if d is not None:
                    d.wait()
            return 0

        lax.fori_loop(0, _NB2 // _GRP2, group, 0)
        plsc.subcore_barrier()
        pltpu.sync_copy(acc.at[pl.ds(s * _RPS, _RPS), :],
                        ou4.at[pl.ds(qn + s * _RPS, _RPS), :])
        plsc.subcore_barrier()
        if qi == 0:
            pltpu.sync_copy(zer, acc.at[pl.ds(s * _RPS, _RPS), :])
            plsc.subcore_barrier()


def _sc_pass2(srcp, dstp, eeT, xp4, zeros_acc):
    return pl.kernel(
        _sc_pass2_body,
        out_type=jax.ShapeDtypeStruct((_NQ * _N, _CW), jnp.float32),
        mesh=_get_mesh(),
        compiler_params=_SC_PARAMS,
        scratch_types=[
            pltpu.VMEM((3, _B), jnp.int32),
            pltpu.VMEM((3, _B), jnp.int32),
            pltpu.VMEM((2, _B), jnp.int32),
            pltpu.VMEM((3, 2, _B), jnp.float32),
            pltpu.VMEM((2, _B, _CW), jnp.float32),
            pltpu.VMEM_SHARED((_N, _CW), jnp.float32),
            pltpu.SemaphoreType.DMA,
            pltpu.SemaphoreType.DMA,
            pltpu.SemaphoreType.DMA,
        ],
    )(srcp, dstp, eeT, xp4, zeros_acc)


def kernel(z, edge_index, W1, a_src1, a_dst1, b1, W2, a_src2, a_dst2, b2,
           Wlin, blin):
    ei = edge_index.astype(jnp.int32)
    pad = jnp.zeros((_EPAD - _E,), jnp.int32)
    srcp = jnp.concatenate([ei[0], pad]).reshape(_EPAD // _B, _B)
    dstp = jnp.concatenate([ei[1], pad]).reshape(_EPAD // _B, _B)
    zeros_n8 = jnp.zeros((_N, _H), jnp.float32)
    zeros_acc = jnp.zeros((_RPS, _CW), jnp.float32)
    b1r = b1.reshape(1, _D)
    b2r = b2.reshape(1, _D)
    blinr = blin.reshape(1, 1)

    Ts1w, Td1w, Ts2w, Td2w = _fold_weights(W1, a_src1, a_dst1, W2, a_src2, a_dst2)

    xp4_1, Ts1, Td1 = _layer1_tc(z, W1, Ts1w, Td1w)
    eeT1, dparts1 = _sc_pass1(srcp, dstp, Ts1, Td1, zeros_n8)
    ou4_1 = _sc_pass2(srcp, dstp, eeT1, xp4_1, zeros_acc)

    xp4_2, Ts2, Td2 = _layer2_tc(ou4_1, xp4_1, dparts1, Ts1, Td1, b1r,
                                 W2, Ts2w, Td2w)
    eeT2, dparts2 = _sc_pass1(srcp, dstp, Ts2, Td2, zeros_n8)
    ou4_2 = _sc_pass2(srcp, dstp, eeT2, xp4_2, zeros_acc)

    return _layer3_tc(ou4_2, xp4_2, dparts2, Ts2, Td2, b2r, Wlin, blinr)


# speedup vs baseline: 1.7329x; 1.0064x over previous
import functools

import jax
import jax.numpy as jnp
from jax import lax
from jax.experimental import pallas as pl
from jax.experimental.pallas import tpu as pltpu
from jax.experimental.pallas import tpu_sc as plsc

_N = 10000
_E = 160000
_EMB = 256
_H = 8
_HID = 64
_D = _H * _HID
_NQ = 4
_CW = 128
_BN = 1000
_NT = _N // _BN

_NWORK = 32
_EPAD = 163840
_B = 128
_EPT1 = _EPAD // _NWORK
_NB1 = _EPT1 // _B
_EPT2 = _EPAD // 16
_NB2 = _EPT2 // _B
_RPS = _N // 16

_SC_PARAMS = pltpu.CompilerParams(needs_layout_passes=False,
                                  use_tc_tiling_on_sc=False)


@functools.cache
def _get_mesh():
    return plsc.VectorSubcoreMesh(core_axis_name="c", subcore_axis_name="s",
                                  num_cores=2, num_subcores=16)


def _fold_body(w1, as1, ad1, w2, as2, ad2, ts1, td1, ts2, td2):
    def fold(w, a, o):
        k = w.shape[0]
        f = jnp.sum(w[...].reshape(k, _H, _HID) * a[...][None], axis=-1)
        o[...] = jnp.concatenate([f, jnp.zeros_like(f)], axis=1)
    fold(w1, as1, ts1)
    fold(w1, ad1, td1)
    fold(w2, as2, ts2)
    fold(w2, ad2, td2)


def _fold_weights(W1, a_src1, a_dst1, W2, a_src2, a_dst2):
    return pl.pallas_call(
        _fold_body,
        out_shape=[jax.ShapeDtypeStruct((_EMB, 16), jnp.float32),
                   jax.ShapeDtypeStruct((_EMB, 16), jnp.float32),
                   jax.ShapeDtypeStruct((_D, 16), jnp.float32),
                   jax.ShapeDtypeStruct((_D, 16), jnp.float32)],
    )(W1, a_src1, a_dst1, W2, a_src2, a_dst2)


def _l1_body(z, wq, ws, wd, xp, ts, td):
    zb = z[...]
    xp[...] = jnp.dot(zb, wq[...], preferred_element_type=jnp.float32)
    ts[...] = jnp.dot(zb, ws[...], preferred_element_type=jnp.float32)
    td[...] = jnp.dot(zb, wd[...], preferred_element_type=jnp.float32)


def _layer1_tc(z, W1, Ws1, Wd1):
    return pl.pallas_call(
        _l1_body,
        grid=(_NT, _NQ),
        in_specs=[
            pl.BlockSpec((_BN, _EMB), lambda i, q: (i, 0)),
            pl.BlockSpec((_EMB, _CW), lambda i, q: (0, q)),
            pl.BlockSpec((_EMB, 16), lambda i, q: (0, 0)),
            pl.BlockSpec((_EMB, 16), lambda i, q: (0, 0)),
        ],
        out_specs=[
            pl.BlockSpec((_BN, _CW), lambda i, q: (q * _NT + i, 0)),
            pl.BlockSpec((_BN, 16), lambda i, q: (i, 0)),
            pl.BlockSpec((_BN, 16), lambda i, q: (i, 0)),
        ],
        out_shape=[jax.ShapeDtypeStruct((_NQ * _N, _CW), jnp.float32),
                   jax.ShapeDtypeStruct((_N, 16), jnp.float32),
                   jax.ShapeDtypeStruct((_N, 16), jnp.float32)],
        compiler_params=pltpu.CompilerParams(
            dimension_semantics=("arbitrary", "arbitrary")),
    )(z, W1, Ws1, Wd1)


def _finalize_block(ou_refs, xp_refs, dp_ref, ts_ref, td_ref, b_ref):
    dsum = jnp.sum(dp_ref[...], axis=0)
    al = ts_ref[...][:, :_H] + td_ref[...][:, :_H]
    el = jnp.exp(jnp.maximum(al, 0.2 * al))
    dtot = dsum + el + 1e-16
    bb = b_ref[...]
    parts = []
    for q in range(_NQ):
        ou = ou_refs[q][...]
        xp = xp_refs[q][...]
        e2 = jnp.concatenate(
            [jnp.broadcast_to(el[:, 2 * q:2 * q + 1], (_BN, _HID)),
             jnp.broadcast_to(el[:, 2 * q + 1:2 * q + 2], (_BN, _HID))], axis=1)
        d2 = jnp.concatenate(
            [jnp.broadcast_to(dtot[:, 2 * q:2 * q + 1], (_BN, _HID)),
             jnp.broadcast_to(dtot[:, 2 * q + 1:2 * q + 2], (_BN, _HID))], axis=1)
        parts.append(jnp.tanh((ou + e2 * xp) / d2 + bb[:, q * _CW:(q + 1) * _CW]))
    return jnp.concatenate(parts, axis=1)


def _l2_body(ou0, ou1, ou2, ou3, xq0, xq1, xq2, xq3, dp, ts, td, b, wq, ws, wd,
             xp2, ts2, td2):
    x2 = _finalize_block((ou0, ou1, ou2, ou3), (xq0, xq1, xq2, xq3),
                         dp, ts, td, b)
    xp2[...] = jnp.dot(x2, wq[...], preferred_element_type=jnp.float32)
    ts2[...] = jnp.dot(x2, ws[...], preferred_element_type=jnp.float32)
    td2[...] = jnp.dot(x2, wd[...], preferred_element_type=jnp.float32)


def _layer2_tc(ou4, xp4, dparts, Ts1, Td1, b1r, W2, Ws2, Wd2):
    chunk = lambda q: pl.BlockSpec((_BN, _CW), lambda i, co, q=q: (q * _NT + i, 0))
    return pl.pallas_call(
        _l2_body,
        grid=(_NT, _NQ),
        in_specs=[chunk(0), chunk(1), chunk(2), chunk(3),
                  chunk(0), chunk(1), chunk(2), chunk(3),
                  pl.BlockSpec((_NWORK, _BN, _H), lambda i, co: (0, i, 0)),
                  pl.BlockSpec((_BN, 16), lambda i, co: (i, 0)),
                  pl.BlockSpec((_BN, 16), lambda i, co: (i, 0)),
                  pl.BlockSpec((1, _D), lambda i, co: (0, 0)),
                  pl.BlockSpec((_D, _CW), lambda i, co: (0, co)),
                  pl.BlockSpec((_D, 16), lambda i, co: (0, 0)),
                  pl.BlockSpec((_D, 16), lambda i, co: (0, 0))],
        out_specs=[
            pl.BlockSpec((_BN, _CW), lambda i, co: (co * _NT + i, 0)),
            pl.BlockSpec((_BN, 16), lambda i, co: (i, 0)),
            pl.BlockSpec((_BN, 16), lambda i, co: (i, 0)),
        ],
        out_shape=[jax.ShapeDtypeStruct((_NQ * _N, _CW), jnp.float32),
                   jax.ShapeDtypeStruct((_N, 16), jnp.float32),
                   jax.ShapeDtypeStruct((_N, 16), jnp.float32)],
        compiler_params=pltpu.CompilerParams(
            dimension_semantics=("arbitrary", "arbitrary")),
    )(ou4, ou4, ou4, ou4, xp4, xp4, xp4, xp4, dparts, Ts1, Td1, b1r, W2, Ws2, Wd2)


def _l3_body(ou0, ou1, ou2, ou3, xq0, xq1, xq2, xq3, dp, ts, td, b, wl, bl, out):
    x3 = _finalize_block((ou0, ou1, ou2, ou3), (xq0, xq1, xq2, xq3),
                         dp, ts, td, b)
    out[...] = jnp.dot(x3, wl[...], preferred_element_type=jnp.float32) + bl[...]


def _layer3_tc(ou4, xp4, dparts, Ts2, Td2, b2r, Wlin, blinr):
    chunk = lambda q: pl.BlockSpec((_BN, _CW), lambda i, q=q: (q * _NT + i, 0))
    return pl.pallas_call(
        _l3_body,
        grid=(_NT,),
        in_specs=[chunk(0), chunk(1), chunk(2), chunk(3),
                  chunk(0), chunk(1), chunk(2), chunk(3),
                  pl.BlockSpec((_NWORK, _BN, _H), lambda i: (0, i, 0)),
                  pl.BlockSpec((_BN, 16), lambda i: (i, 0)),
                  pl.BlockSpec((_BN, 16), lambda i: (i, 0)),
                  pl.BlockSpec((1, _D), lambda i: (0, 0)),
                  pl.BlockSpec((_D, 1), lambda i: (0, 0)),
                  pl.BlockSpec((1, 1), lambda i: (0, 0))],
        out_specs=pl.BlockSpec((_BN, 1), lambda i: (i, 0)),
        out_shape=jax.ShapeDtypeStruct((_N, 1), jnp.float32),
        compiler_params=pltpu.CompilerParams(
            dimension_semantics=("arbitrary",)),
    )(ou4, ou4, ou4, ou4, xp4, xp4, xp4, xp4, dparts, Ts2, Td2, b2r, Wlin, blinr)


def _sc_pass1_body(srcp, dstp, ts, td, zer, eeT, dparts,
                   src_v, dst_v, g1, g2, eeb, den_v, gsem, ssem):
    w = lax.axis_index("s") * 2 + lax.axis_index("c")
    lane = lax.iota(jnp.int32, 16)
    m8 = lane < 8
    pltpu.sync_copy(zer, den_v)
    pltpu.sync_copy(srcp.at[pl.ds(w * _NB1, _NB1), :], src_v)
    pltpu.sync_copy(dstp.at[pl.ds(w * _NB1, _NB1), :], dst_v)

    def group(g, _):
        def gathers(b, slot):
            j = g * _GRP + b
            return (pltpu.async_copy(ts.at[src_v.at[j]], g1.at[slot], gsem),
                    pltpu.async_copy(td.at[dst_v.at[j]], g2.at[slot], gsem))

        pend_g = gathers(0, 0)
        pend_s = [None, None]
        for b in range(_GRP):
            sl = b % 2
            j = g * _GRP + b
            for d in pend_g:
                d.wait()
            if b + 1 < _GRP:
                pend_g = gathers(b + 1, (b + 1) % 2)
            if pend_s[sl] is not None:
                for d in pend_s[sl]:
                    d.wait()
                pend_s[sl] = None
            base = (w * _NB1 + j) * _B
            j16 = jnp.full((16,), j, jnp.int32)
            s16 = jnp.full((16,), sl, jnp.int32)

            def edge(k, _):
                k16 = jnp.full((16,), k, jnp.int32)
                v = g1[sl, k, :] + g2[sl, k, :]
                ee = jnp.exp(jnp.maximum(v, 0.2 * v))
                ee = jnp.where(base + k < _E, ee, 0.0)
                dk = plsc.load_gather(dst_v, [j16, k16])
                plsc.addupdate_scatter(den_v, [dk, lane], ee, mask=m8)
                plsc.store_scatter(eeb, [s16, lane * _B + k16], ee, mask=m8)
                return 0

            lax.fori_loop(0, _B, edge, 0)
            pend_s[sl] = tuple(
                pltpu.async_copy(eeb.at[sl, pl.ds(h * _B, _B)],
                                 eeT.at[h, w * _NB1 + j], ssem)
                for h in range(_H))
        for p in pend_s:
            if p is not None:
                for d in p:
                    d.wait()
        return 0

    lax.fori_loop(0, _NB1 // _GRP, group, 0)
    pltpu.sync_copy(den_v, dparts.at[w])


def _sc_pass1(srcp, dstp, Ts, Td, zeros_n8):
    return pl.kernel(
        _sc_pass1_body,
        out_type=[jax.ShapeDtypeStruct((_H, _EPAD // _B, _B), jnp.float32),
                  jax.ShapeDtypeStruct((_NWORK, _N, _H), jnp.float32)],
        mesh=_get_mesh(),
        compiler_params=_SC_PARAMS,
        scratch_types=[
            pltpu.VMEM((_NB1, _B), jnp.int32),
            pltpu.VMEM((_NB1, _B), jnp.int32),
            pltpu.VMEM((2, _B, 16), jnp.float32),
            pltpu.VMEM((2, _B, 16), jnp.float32),
            pltpu.VMEM((2, _H * _B), jnp.float32),
            pltpu.VMEM((_N, _H), jnp.float32),
            pltpu.SemaphoreType.DMA,
            pltpu.SemaphoreType.DMA,
        ],
    )(srcp, dstp, Ts, Td, zeros_n8)


_GRP = 40
_GRP2 = 40


def _sc_pass2_body(srcp, dstp, eeT, xp4, zer, ou4,
                   src_v, dst_v, idx_v, eeb, rows,
                   acc, lsem, gsem, ssem):
    c = lax.axis_index("c")
    s = lax.axis_index("s")
    pltpu.sync_copy(zer, acc.at[pl.ds(s * _RPS, _RPS), :])
    plsc.subcore_barrier()
    for qi in range(2):
        q = 2 * c + qi
        qn = q * _N

        def group(g, _):
            row0 = s * _NB2 + g * _GRP2

            def loads(b):
                r = row0 + b
                return (pltpu.async_copy(srcp.at[r], src_v.at[b % 3], lsem),
                        pltpu.async_copy(dstp.at[r], dst_v.at[b % 3], lsem),
                        pltpu.async_copy(eeT.at[pl.ds(2 * q, 2), r],
                                         eeb.at[b % 3], lsem))

            def build_and_gather(b):
                for l in range(_B // 16):
                    idx_v[b % 2, pl.ds(l * 16, 16)] = (
                        src_v[b % 3, pl.ds(l * 16, 16)] + qn)
                return pltpu.async_copy(xp4.at[idx_v.at[b % 2]],
                                        rows.at[b % 2], gsem)

            pend_l = loads(0)
            for d in pend_l:
                d.wait()
            pend_g = [build_and_gather(0), None]
            pend_l = loads(1)
            pend_s = [None, None]
            for b in range(_GRP2):
                s2, s3 = b % 2, b % 3
                n2 = (b + 1) % 2
                if b + 1 < _GRP2:
                    for d in pend_l:
                        d.wait()
                    if pend_s[n2] is not None:
                        pend_s[n2].wait()
                        pend_s[n2] = None
                    pend_g[n2] = build_and_gather(b + 1)
                    if b + 2 < _GRP2:
                        pend_l = loads(b + 2)
                pend_g[s2].wait()
                pend_g[s2] = None

                s16 = jnp.full((16,), s3, jnp.int32)
                h0 = jnp.zeros((16,), jnp.int32)
                h1 = jnp.ones((16,), jnp.int32)

                def edge(i, _):
                    k = 2 * i
                    k16 = jnp.full((16,), k, jnp.int32)
                    e0a = plsc.load_gather(eeb, [s16, h0, k16])
                    e1a = plsc.load_gather(eeb, [s16, h1, k16])
                    e0b = plsc.load_gather(eeb, [s16, h0, k16 + 1])
                    e1b = plsc.load_gather(eeb, [s16, h1, k16 + 1])
                    for l in range(4):
                        rows[s2, k, pl.ds(l * 16, 16)] = (
                            rows[s2, k, pl.ds(l * 16, 16)] * e0a)
                        rows[s2, k + 1, pl.ds(l * 16, 16)] = (
                            rows[s2, k + 1, pl.ds(l * 16, 16)] * e0b)
                    for l in range(4, 8):
                        rows[s2, k, pl.ds(l * 16, 16)] = (
                            rows[s2, k, pl.ds(l * 16, 16)] * e1a)
                        rows[s2, k + 1, pl.ds(l * 16, 16)] = (
                            rows[s2, k + 1, pl.ds(l * 16, 16)] * e1b)
                    return 0

                lax.fori_loop(0, _B // 2, edge, 0)
                pend_s[s2] = pltpu.async_copy(rows.at[s2],
                                              acc.at[dst_v.at[s3]], ssem,
                                              add=True)
            for d in pend_s:
                if d is not None:
                    d.wait()
            return 0

        lax.fori_loop(0, _NB2 // _GRP2, group, 0)
        plsc.subcore_barrier()
        pltpu.sync_copy(acc.at[pl.ds(s * _RPS, _RPS), :],
                        ou4.at[pl.ds(qn + s * _RPS, _RPS), :])
        plsc.subcore_barrier()
        if qi == 0:
            pltpu.sync_copy(zer, acc.at[pl.ds(s * _RPS, _RPS), :])
            plsc.subcore_barrier()


def _sc_pass2(srcp, dstp, eeT, xp4, zeros_acc):
    return pl.kernel(
        _sc_pass2_body,
        out_type=jax.ShapeDtypeStruct((_NQ * _N, _CW), jnp.float32),
        mesh=_get_mesh(),
        compiler_params=_SC_PARAMS,
        scratch_types=[
            pltpu.VMEM((3, _B), jnp.int32),
            pltpu.VMEM((3, _B), jnp.int32),
            pltpu.VMEM((2, _B), jnp.int32),
            pltpu.VMEM((3, 2, _B), jnp.float32),
            pltpu.VMEM((2, _B, _CW), jnp.float32),
            pltpu.VMEM_SHARED((_N, _CW), jnp.float32),
            pltpu.SemaphoreType.DMA,
            pltpu.SemaphoreType.DMA,
            pltpu.SemaphoreType.DMA,
        ],
    )(srcp, dstp, eeT, xp4, zeros_acc)


def kernel(z, edge_index, W1, a_src1, a_dst1, b1, W2, a_src2, a_dst2, b2,
           Wlin, blin):
    ei = edge_index.astype(jnp.int32)
    pad = jnp.zeros((_EPAD - _E,), jnp.int32)
    srcp = jnp.concatenate([ei[0], pad]).reshape(_EPAD // _B, _B)
    dstp = jnp.concatenate([ei[1], pad]).reshape(_EPAD // _B, _B)
    zeros_n8 = jnp.zeros((_N, _H), jnp.float32)
    zeros_acc = jnp.zeros((_RPS, _CW), jnp.float32)
    b1r = b1.reshape(1, _D)
    b2r = b2.reshape(1, _D)
    blinr = blin.reshape(1, 1)

    Ts1w, Td1w, Ts2w, Td2w = _fold_weights(W1, a_src1, a_dst1, W2, a_src2, a_dst2)

    xp4_1, Ts1, Td1 = _layer1_tc(z, W1, Ts1w, Td1w)
    eeT1, dparts1 = _sc_pass1(srcp, dstp, Ts1, Td1, zeros_n8)
    ou4_1 = _sc_pass2(srcp, dstp, eeT1, xp4_1, zeros_acc)

    xp4_2, Ts2, Td2 = _layer2_tc(ou4_1, xp4_1, dparts1, Ts1, Td1, b1r,
                                 W2, Ts2w, Td2w)
    eeT2, dparts2 = _sc_pass1(srcp, dstp, Ts2, Td2, zeros_n8)
    ou4_2 = _sc_pass2(srcp, dstp, eeT2, xp4_2, zeros_acc)

    return _layer3_tc(ou4_2, xp4_2, dparts2, Ts2, Td2, b2r, Wlin, blinr)


# speedup vs baseline: 1.7523x; 1.0112x over previous
import functools

import jax
import jax.numpy as jnp
from jax import lax
from jax.experimental import pallas as pl
from jax.experimental.pallas import tpu as pltpu
from jax.experimental.pallas import tpu_sc as plsc

_N = 10000
_E = 160000
_EMB = 256
_H = 8
_HID = 64
_D = _H * _HID
_NQ = 4
_CW = 128
_BN = 1000
_NT = _N // _BN

_NWORK = 32
_EPAD = 163840
_B = 128
_EPT1 = _EPAD // _NWORK
_NB1 = _EPT1 // _B
_EPT2 = _EPAD // 16
_NB2 = _EPT2 // _B
_RPS = _N // 16

_SC_PARAMS = pltpu.CompilerParams(needs_layout_passes=False,
                                  use_tc_tiling_on_sc=False)


@functools.cache
def _get_mesh():
    return plsc.VectorSubcoreMesh(core_axis_name="c", subcore_axis_name="s",
                                  num_cores=2, num_subcores=16)


def _fold_body(w1, as1, ad1, w2, as2, ad2, ts1, td1, ts2, td2):
    def fold(w, a, o):
        k = w.shape[0]
        f = jnp.sum(w[...].reshape(k, _H, _HID) * a[...][None], axis=-1)
        o[...] = jnp.concatenate([f, jnp.zeros_like(f)], axis=1)
    fold(w1, as1, ts1)
    fold(w1, ad1, td1)
    fold(w2, as2, ts2)
    fold(w2, ad2, td2)


def _fold_weights(W1, a_src1, a_dst1, W2, a_src2, a_dst2):
    return pl.pallas_call(
        _fold_body,
        out_shape=[jax.ShapeDtypeStruct((_EMB, 16), jnp.float32),
                   jax.ShapeDtypeStruct((_EMB, 16), jnp.float32),
                   jax.ShapeDtypeStruct((_D, 16), jnp.float32),
                   jax.ShapeDtypeStruct((_D, 16), jnp.float32)],
    )(W1, a_src1, a_dst1, W2, a_src2, a_dst2)


def _l1_body(z, wq, ws, wd, xp, ts, td):
    zb = z[...]
    xp[...] = jnp.dot(zb, wq[...], preferred_element_type=jnp.float32)
    ts[...] = jnp.dot(zb, ws[...], preferred_element_type=jnp.float32)
    td[...] = jnp.dot(zb, wd[...], preferred_element_type=jnp.float32)


def _layer1_tc(z, W1, Ws1, Wd1):
    return pl.pallas_call(
        _l1_body,
        grid=(_NT, _NQ),
        in_specs=[
            pl.BlockSpec((_BN, _EMB), lambda i, q: (i, 0)),
            pl.BlockSpec((_EMB, _CW), lambda i, q: (0, q)),
            pl.BlockSpec((_EMB, 16), lambda i, q: (0, 0)),
            pl.BlockSpec((_EMB, 16), lambda i, q: (0, 0)),
        ],
        out_specs=[
            pl.BlockSpec((_BN, _CW), lambda i, q: (q * _NT + i, 0)),
            pl.BlockSpec((_BN, 16), lambda i, q: (i, 0)),
            pl.BlockSpec((_BN, 16), lambda i, q: (i, 0)),
        ],
        out_shape=[jax.ShapeDtypeStruct((_NQ * _N, _CW), jnp.float32),
                   jax.ShapeDtypeStruct((_N, 16), jnp.float32),
                   jax.ShapeDtypeStruct((_N, 16), jnp.float32)],
        compiler_params=pltpu.CompilerParams(
            dimension_semantics=("arbitrary", "arbitrary")),
    )(z, W1, Ws1, Wd1)


def _finalize_block(ou_refs, xp_refs, dp_ref, ts_ref, td_ref, b_ref):
    dsum = jnp.sum(dp_ref[...], axis=0)
    al = ts_ref[...][:, :_H] + td_ref[...][:, :_H]
    el = jnp.exp(jnp.maximum(al, 0.2 * al))
    dtot = dsum + el + 1e-16
    bb = b_ref[...]
    parts = []
    for q in range(_NQ):
        ou = ou_refs[q][...]
        xp = xp_refs[q][...]
        e2 = jnp.concatenate(
            [jnp.broadcast_to(el[:, 2 * q:2 * q + 1], (_BN, _HID)),
             jnp.broadcast_to(el[:, 2 * q + 1:2 * q + 2], (_BN, _HID))], axis=1)
        d2 = jnp.concatenate(
            [jnp.broadcast_to(dtot[:, 2 * q:2 * q + 1], (_BN, _HID)),
             jnp.broadcast_to(dtot[:, 2 * q + 1:2 * q + 2], (_BN, _HID))], axis=1)
        parts.append(jnp.tanh((ou + e2 * xp) / d2 + bb[:, q * _CW:(q + 1) * _CW]))
    return jnp.concatenate(parts, axis=1)


def _l2_body(ou0, ou1, ou2, ou3, xq0, xq1, xq2, xq3, dp, ts, td, b, wq, ws, wd,
             xp2, ts2, td2):
    x2 = _finalize_block((ou0, ou1, ou2, ou3), (xq0, xq1, xq2, xq3),
                         dp, ts, td, b)
    xp2[...] = jnp.dot(x2, wq[...], preferred_element_type=jnp.float32)
    ts2[...] = jnp.dot(x2, ws[...], preferred_element_type=jnp.float32)
    td2[...] = jnp.dot(x2, wd[...], preferred_element_type=jnp.float32)


def _layer2_tc(ou4, xp4, dparts, Ts1, Td1, b1r, W2, Ws2, Wd2):
    chunk = lambda q: pl.BlockSpec((_BN, _CW), lambda i, co, q=q: (q * _NT + i, 0))
    return pl.pallas_call(
        _l2_body,
        grid=(_NT, _NQ),
        in_specs=[chunk(0), chunk(1), chunk(2), chunk(3),
                  chunk(0), chunk(1), chunk(2), chunk(3),
                  pl.BlockSpec((_NWORK, _BN, _H), lambda i, co: (0, i, 0)),
                  pl.BlockSpec((_BN, 16), lambda i, co: (i, 0)),
                  pl.BlockSpec((_BN, 16), lambda i, co: (i, 0)),
                  pl.BlockSpec((1, _D), lambda i, co: (0, 0)),
                  pl.BlockSpec((_D, _CW), lambda i, co: (0, co)),
                  pl.BlockSpec((_D, 16), lambda i, co: (0, 0)),
                  pl.BlockSpec((_D, 16), lambda i, co: (0, 0))],
        out_specs=[
            pl.BlockSpec((_BN, _CW), lambda i, co: (co * _NT + i, 0)),
            pl.BlockSpec((_BN, 16), lambda i, co: (i, 0)),
            pl.BlockSpec((_BN, 16), lambda i, co: (i, 0)),
        ],
        out_shape=[jax.ShapeDtypeStruct((_NQ * _N, _CW), jnp.float32),
                   jax.ShapeDtypeStruct((_N, 16), jnp.float32),
                   jax.ShapeDtypeStruct((_N, 16), jnp.float32)],
        compiler_params=pltpu.CompilerParams(
            dimension_semantics=("arbitrary", "arbitrary")),
    )(ou4, ou4, ou4, ou4, xp4, xp4, xp4, xp4, dparts, Ts1, Td1, b1r, W2, Ws2, Wd2)


def _l3_body(ou0, ou1, ou2, ou3, xq0, xq1, xq2, xq3, dp, ts, td, b, wl, bl, out):
    x3 = _finalize_block((ou0, ou1, ou2, ou3), (xq0, xq1, xq2, xq3),
                         dp, ts, td, b)
    out[...] = jnp.dot(x3, wl[...], preferred_element_type=jnp.float32) + bl[...]


def _layer3_tc(ou4, xp4, dparts, Ts2, Td2, b2r, Wlin, blinr):
    chunk = lambda q: pl.BlockSpec((_BN, _CW), lambda i, q=q: (q * _NT + i, 0))
    return pl.pallas_call(
        _l3_body,
        grid=(_NT,),
        in_specs=[chunk(0), chunk(1), chunk(2), chunk(3),
                  chunk(0), chunk(1), chunk(2), chunk(3),
                  pl.BlockSpec((_NWORK, _BN, _H), lambda i: (0, i, 0)),
                  pl.BlockSpec((_BN, 16), lambda i: (i, 0)),
                  pl.BlockSpec((_BN, 16), lambda i: (i, 0)),
                  pl.BlockSpec((1, _D), lambda i: (0, 0)),
                  pl.BlockSpec((_D, 1), lambda i: (0, 0)),
                  pl.BlockSpec((1, 1), lambda i: (0, 0))],
        out_specs=pl.BlockSpec((_BN, 1), lambda i: (i, 0)),
        out_shape=jax.ShapeDtypeStruct((_N, 1), jnp.float32),
        compiler_params=pltpu.CompilerParams(
            dimension_semantics=("arbitrary",)),
    )(ou4, ou4, ou4, ou4, xp4, xp4, xp4, xp4, dparts, Ts2, Td2, b2r, Wlin, blinr)


def _sc_pass1_body(srcp, dstp, ts, td, zer, eeT, dparts,
                   src_v, dst_v, g1, g2, eeb, den_v, gsem, ssem):
    w = lax.axis_index("s") * 2 + lax.axis_index("c")
    lane = lax.iota(jnp.int32, 16)
    m8 = lane < 8
    pltpu.sync_copy(zer, den_v)
    pltpu.sync_copy(srcp.at[pl.ds(w * _NB1, _NB1), :], src_v)
    pltpu.sync_copy(dstp.at[pl.ds(w * _NB1, _NB1), :], dst_v)

    def group(g, _):
        def gathers(b, slot):
            j = g * _GRP + b
            return (pltpu.async_copy(ts.at[src_v.at[j]], g1.at[slot], gsem),
                    pltpu.async_copy(td.at[dst_v.at[j]], g2.at[slot], gsem))

        pend_g = gathers(0, 0)
        pend_s = [None, None]
        for b in range(_GRP):
            sl = b % 2
            j = g * _GRP + b
            for d in pend_g:
                d.wait()
            if b + 1 < _GRP:
                pend_g = gathers(b + 1, (b + 1) % 2)
            if pend_s[sl] is not None:
                for d in pend_s[sl]:
                    d.wait()
                pend_s[sl] = None
            base = (w * _NB1 + j) * _B
            j16 = jnp.full((16,), j, jnp.int32)
            s16 = jnp.full((16,), sl, jnp.int32)

            def edge(k, _):
                k16 = jnp.full((16,), k, jnp.int32)
                v = g1[sl, k, :] + g2[sl, k, :]
                ee = jnp.exp(jnp.maximum(v, 0.2 * v))
                ee = jnp.where(base + k < _E, ee, 0.0)
                dk = plsc.load_gather(dst_v, [j16, k16])
                plsc.addupdate_scatter(den_v, [dk, lane], ee, mask=m8)
                plsc.store_scatter(eeb, [s16, lane * _B + k16], ee, mask=m8)
                return 0

            lax.fori_loop(0, _B, edge, 0)
            pend_s[sl] = tuple(
                pltpu.async_copy(eeb.at[sl, pl.ds(h * _B, _B)],
                                 eeT.at[h, w * _NB1 + j], ssem)
                for h in range(_H))
        for p in pend_s:
            if p is not None:
                for d in p:
                    d.wait()
        return 0

    lax.fori_loop(0, _NB1 // _GRP, group, 0)
    pltpu.sync_copy(den_v, dparts.at[w])


def _sc_pass1(srcp, dstp, Ts, Td, zeros_n8):
    return pl.kernel(
        _sc_pass1_body,
        out_type=[jax.ShapeDtypeStruct((_H, _EPAD // _B, _B), jnp.float32),
                  jax.ShapeDtypeStruct((_NWORK, _N, _H), jnp.float32)],
        mesh=_get_mesh(),
        compiler_params=_SC_PARAMS,
        scratch_types=[
            pltpu.VMEM((_NB1, _B), jnp.int32),
            pltpu.VMEM((_NB1, _B), jnp.int32),
            pltpu.VMEM((2, _B, 16), jnp.float32),
            pltpu.VMEM((2, _B, 16), jnp.float32),
            pltpu.VMEM((2, _H * _B), jnp.float32),
            pltpu.VMEM((_N, _H), jnp.float32),
            pltpu.SemaphoreType.DMA,
            pltpu.SemaphoreType.DMA,
        ],
    )(srcp, dstp, Ts, Td, zeros_n8)


_GRP = 40
_GRP2 = 40


def _sc_pass2_body(srcp, dstp, eeT, xp4, zer, ou4,
                   src_v, dst_v, idx_v, eeb, rows,
                   acc, lsem, gsem, ssem):
    c = lax.axis_index("c")
    s = lax.axis_index("s")
    pltpu.sync_copy(zer, acc.at[pl.ds(s * _RPS, _RPS), :])
    plsc.subcore_barrier()
    for qi in range(2):
        q = 2 * c + qi
        qn = q * _N

        def group(g, _):
            row0 = s * _NB2 + g * _GRP2

            def loads(b):
                r = row0 + b
                return (pltpu.async_copy(srcp.at[r], src_v.at[b % 2], lsem),
                        pltpu.async_copy(dstp.at[r], dst_v.at[b % 4], lsem),
                        pltpu.async_copy(eeT.at[pl.ds(2 * q, 2), r],
                                         eeb.at[b % 3], lsem))

            def build_and_gather(b):
                for l in range(_B // 16):
                    idx_v[b % 2, pl.ds(l * 16, 16)] = (
                        src_v[b % 2, pl.ds(l * 16, 16)] + qn)
                return pltpu.async_copy(xp4.at[idx_v.at[b % 2]],
                                        rows.at[b % 3], gsem)

            pend_l = loads(0)
            for d in pend_l:
                d.wait()
            pend_g = [build_and_gather(0), None, None]
            pend_l = loads(1)
            pend_s = [None, None, None]
            for b in range(_GRP2):
                s2, s3 = b % 3, b % 3
                n2 = (b + 1) % 3
                if b + 1 < _GRP2:
                    for d in pend_l:
                        d.wait()
                    if pend_s[n2] is not None:
                        pend_s[n2].wait()
                        pend_s[n2] = None
                    pend_g[n2] = build_and_gather(b + 1)
                    if b + 2 < _GRP2:
                        pend_l = loads(b + 2)
                pend_g[s2].wait()
                pend_g[s2] = None

                s16 = jnp.full((16,), s3, jnp.int32)
                h0 = jnp.zeros((16,), jnp.int32)
                h1 = jnp.ones((16,), jnp.int32)

                def edge(i, _):
                    k = 2 * i
                    k16 = jnp.full((16,), k, jnp.int32)
                    e0a = plsc.load_gather(eeb, [s16, h0, k16])
                    e1a = plsc.load_gather(eeb, [s16, h1, k16])
                    e0b = plsc.load_gather(eeb, [s16, h0, k16 + 1])
                    e1b = plsc.load_gather(eeb, [s16, h1, k16 + 1])
                    for l in range(4):
                        rows[s2, k, pl.ds(l * 16, 16)] = (
                            rows[s2, k, pl.ds(l * 16, 16)] * e0a)
                        rows[s2, k + 1, pl.ds(l * 16, 16)] = (
                            rows[s2, k + 1, pl.ds(l * 16, 16)] * e0b)
                    for l in range(4, 8):
                        rows[s2, k, pl.ds(l * 16, 16)] = (
                            rows[s2, k, pl.ds(l * 16, 16)] * e1a)
                        rows[s2, k + 1, pl.ds(l * 16, 16)] = (
                            rows[s2, k + 1, pl.ds(l * 16, 16)] * e1b)
                    return 0

                lax.fori_loop(0, _B // 2, edge, 0)
                pend_s[s2] = pltpu.async_copy(rows.at[s2],
                                              acc.at[dst_v.at[b % 4]], ssem,
                                              add=True)
            for d in pend_s:
                if d is not None:
                    d.wait()
            return 0

        lax.fori_loop(0, _NB2 // _GRP2, group, 0)
        plsc.subcore_barrier()
        pltpu.sync_copy(acc.at[pl.ds(s * _RPS, _RPS), :],
                        ou4.at[pl.ds(qn + s * _RPS, _RPS), :])
        plsc.subcore_barrier()
        if qi == 0:
            pltpu.sync_copy(zer, acc.at[pl.ds(s * _RPS, _RPS), :])
            plsc.subcore_barrier()


def _sc_pass2(srcp, dstp, eeT, xp4, zeros_acc):
    return pl.kernel(
        _sc_pass2_body,
        out_type=jax.ShapeDtypeStruct((_NQ * _N, _CW), jnp.float32),
        mesh=_get_mesh(),
        compiler_params=_SC_PARAMS,
        scratch_types=[
            pltpu.VMEM((2, _B), jnp.int32),
            pltpu.VMEM((4, _B), jnp.int32),
            pltpu.VMEM((2, _B), jnp.int32),
            pltpu.VMEM((3, 2, _B), jnp.float32),
            pltpu.VMEM((3, _B, _CW), jnp.float32),
            pltpu.VMEM_SHARED((_N, _CW), jnp.float32),
            pltpu.SemaphoreType.DMA,
            pltpu.SemaphoreType.DMA,
            pltpu.SemaphoreType.DMA,
        ],
    )(srcp, dstp, eeT, xp4, zeros_acc)


def kernel(z, edge_index, W1, a_src1, a_dst1, b1, W2, a_src2, a_dst2, b2,
           Wlin, blin):
    ei = edge_index.astype(jnp.int32)
    pad = jnp.zeros((_EPAD - _E,), jnp.int32)
    srcp = jnp.concatenate([ei[0], pad]).reshape(_EPAD // _B, _B)
    dstp = jnp.concatenate([ei[1], pad]).reshape(_EPAD // _B, _B)
    zeros_n8 = jnp.zeros((_N, _H), jnp.float32)
    zeros_acc = jnp.zeros((_RPS, _CW), jnp.float32)
    b1r = b1.reshape(1, _D)
    b2r = b2.reshape(1, _D)
    blinr = blin.reshape(1, 1)

    Ts1w, Td1w, Ts2w, Td2w = _fold_weights(W1, a_src1, a_dst1, W2, a_src2, a_dst2)

    xp4_1, Ts1, Td1 = _layer1_tc(z, W1, Ts1w, Td1w)
    eeT1, dparts1 = _sc_pass1(srcp, dstp, Ts1, Td1, zeros_n8)
    ou4_1 = _sc_pass2(srcp, dstp, eeT1, xp4_1, zeros_acc)

    xp4_2, Ts2, Td2 = _layer2_tc(ou4_1, xp4_1, dparts1, Ts1, Td1, b1r,
                                 W2, Ts2w, Td2w)
    eeT2, dparts2 = _sc_pass1(srcp, dstp, Ts2, Td2, zeros_n8)
    ou4_2 = _sc_pass2(srcp, dstp, eeT2, xp4_2, zeros_acc)

    return _layer3_tc(ou4_2, xp4_2, dparts2, Ts2, Td2, b2r, Wlin, blinr)
